# baseline (speedup 1.0000x reference)
import functools

import jax
import jax.numpy as jnp
from jax import lax
from jax.experimental import pallas as pl
from jax.experimental.pallas import tpu as pltpu
from jax.experimental.pallas import tpu_sc as plsc

_NC = 2
_NS = 16
_NW = _NC * _NS

_TB = 256


_SP = 40


def _router_book_body(x_ref, rw_ref, dst_ref, sp_ref):
    t, e = x_ref.shape[0], rw_ref.shape[0]
    logits = lax.dot_general(
        x_ref[...], rw_ref[...], (((1,), (1,)), ((), ())),
        preferred_element_type=jnp.float32,
    )
    m = jnp.max(logits, axis=1, keepdims=True)
    ids = lax.broadcasted_iota(jnp.int32, (t, e), 1)
    picked = jnp.where(logits == m, ids, e)
    eidx = jnp.min(picked, axis=1, keepdims=True)
    oh = (ids == eidx).astype(jnp.int32)
    c = oh
    k = 1
    while k < t:
        c = c + jnp.concatenate(
            [jnp.zeros((k, e), jnp.int32), c[: t - k, :]], axis=0)
        k *= 2
    counts = c[t - 1: t, :]
    nblk = (counts + (_TB - 1)) // _TB
    inc = nblk
    k = 1
    while k < e:
        inc = inc + jnp.concatenate(
            [jnp.zeros((1, k), jnp.int32), inc[:, : e - k]], axis=1)
        k *= 2
    blk_start = inc - nblk
    total = inc[:, e - 1: e]
    rank = jnp.sum((c - 1) * oh, axis=1, keepdims=True)
    dst_ref[...] = (jnp.sum(oh * jnp.broadcast_to(blk_start, (t, e)),
                            axis=1, keepdims=True) * _TB + rank)
    jb = lax.broadcasted_iota(jnp.int32, (_SP, e), 0) - 1
    ge = (jb >= jnp.broadcast_to(blk_start, (_SP, e))).astype(jnp.int32)
    be = jnp.clip(jnp.sum(ge, axis=1, keepdims=True) - 1, 0, e - 1)
    row = lax.broadcasted_iota(jnp.int32, (_SP, 1), 0)
    sp_ref[...] = jnp.where(row == 0,
                            jnp.broadcast_to(total, (_SP, 1)), be)


def _moe_body(sp_ref, xp_ref, wg_ref, wu_ref, wd_ref, out_ref):
    @pl.when(pl.program_id(0) < sp_ref[0, 0])
    def _():
        xb = xp_ref[...]
        g = lax.dot_general(xb, wg_ref[0], (((1,), (1,)), ((), ())),
                            preferred_element_type=jnp.float32)
        u = lax.dot_general(xb, wu_ref[0], (((1,), (1,)), ((), ())),
                            preferred_element_type=jnp.float32)
        hmid = jnp.square(jnp.maximum(g, 0.0)) * u
        out_ref[...] = lax.dot_general(hmid, wd_ref[0],
                                       (((1,), (1,)), ((), ())),
                                       preferred_element_type=jnp.float32)


def kernel(x, router_w, Wg, Wu, Wd):
    b, s, hdim = x.shape
    e, d, _ = Wg.shape
    t = b * s
    xf = x.reshape(t, hdim)

    nb = t // _TB + e
    p = nb * _TB
    dst, sp = pl.pallas_call(
        _router_book_body,
        out_shape=(jax.ShapeDtypeStruct((t, 1), jnp.int32),
                   jax.ShapeDtypeStruct((_SP, 1), jnp.int32)),
    )(xf, router_w)
    bpw = t // _NW
    dst2 = dst.reshape(_NW, bpw)

    mesh = plsc.VectorSubcoreMesh(core_axis_name="c", subcore_axis_name="s")

    @functools.partial(
        pl.kernel, mesh=mesh,
        out_type=jax.ShapeDtypeStruct((p, hdim), jnp.float32),
        scratch_types=[
            pltpu.VMEM((bpw,), jnp.int32),
            pltpu.VMEM((bpw, hdim), jnp.float32),
            pltpu.SemaphoreType.DMA,
        ],
    )
    def _dispatch(xf_hbm, dst_hbm, xp_hbm, idx_v, rows_v, sem):
        wid = lax.axis_index("s") * _NC + lax.axis_index("c")
        pltpu.sync_copy(dst_hbm.at[wid], idx_v)
        pltpu.sync_copy(xf_hbm.at[pl.ds(wid * bpw, bpw)], rows_v)
        pltpu.async_copy(rows_v, xp_hbm.at[idx_v], sem).wait()

    xp = _dispatch(xf, dst2)

    grid_spec = pltpu.PrefetchScalarGridSpec(
        num_scalar_prefetch=1,
        grid=(nb,),
        in_specs=[
            pl.BlockSpec((_TB, hdim),
                         lambda i, sp: (jnp.minimum(i, sp[0, 0] - 1), 0)),
            pl.BlockSpec((1, d, hdim), lambda i, sp: (sp[i + 1, 0], 0, 0)),
            pl.BlockSpec((1, d, hdim), lambda i, sp: (sp[i + 1, 0], 0, 0)),
            pl.BlockSpec((1, hdim, d), lambda i, sp: (sp[i + 1, 0], 0, 0)),
        ],
        out_specs=pl.BlockSpec((_TB, hdim),
                               lambda i, sp: (jnp.minimum(i, sp[0, 0] - 1), 0)),
    )
    out_p = pl.pallas_call(
        _moe_body,
        grid_spec=grid_spec,
        out_shape=jax.ShapeDtypeStruct((p, hdim), jnp.float32),
    )(sp, xp, Wg, Wu, Wd)

    @functools.partial(
        pl.kernel, mesh=mesh,
        out_type=jax.ShapeDtypeStruct((t, hdim), jnp.float32),
        scratch_types=[
            pltpu.VMEM((bpw,), jnp.int32),
            pltpu.VMEM((bpw, hdim), jnp.float32),
            pltpu.SemaphoreType.DMA,
        ],
    )
    def _combine(outp_hbm, dst_hbm, out_hbm, idx_v, rows_v, sem):
        wid = lax.axis_index("s") * _NC + lax.axis_index("c")
        pltpu.sync_copy(dst_hbm.at[wid], idx_v)
        pltpu.async_copy(outp_hbm.at[idx_v], rows_v, sem).wait()
        pltpu.sync_copy(rows_v, out_hbm.at[pl.ds(wid * bpw, bpw)])

    out = _combine(out_p, dst2)
    return out.reshape(b, s, hdim)


# speedup vs baseline: 1.0819x; 1.0819x over previous
import functools

import jax
import jax.numpy as jnp
from jax import lax
from jax.experimental import pallas as pl
from jax.experimental.pallas import tpu as pltpu
from jax.experimental.pallas import tpu_sc as plsc

_NC = 2
_NS = 16
_NW = _NC * _NS

_TB = 384


_SP = 40


def _router_book_body(x_ref, rw_ref, dst_ref, sp_ref):
    t, e = x_ref.shape[0], rw_ref.shape[0]
    logits = lax.dot_general(
        x_ref[...], rw_ref[...], (((1,), (1,)), ((), ())),
        preferred_element_type=jnp.float32,
    )
    m = jnp.max(logits, axis=1, keepdims=True)
    ids = lax.broadcasted_iota(jnp.int32, (t, e), 1)
    picked = jnp.where(logits == m, ids, e)
    eidx = jnp.min(picked, axis=1, keepdims=True)
    oh = (ids == eidx).astype(jnp.int32)
    c = oh
    k = 1
    while k < t:
        c = c + jnp.concatenate(
            [jnp.zeros((k, e), jnp.int32), c[: t - k, :]], axis=0)
        k *= 2
    counts = c[t - 1: t, :]
    nblk = (counts + (_TB - 1)) // _TB
    inc = nblk
    k = 1
    while k < e:
        inc = inc + jnp.concatenate(
            [jnp.zeros((1, k), jnp.int32), inc[:, : e - k]], axis=1)
        k *= 2
    blk_start = inc - nblk
    total = inc[:, e - 1: e]
    rank = jnp.sum((c - 1) * oh, axis=1, keepdims=True)
    dst_ref[...] = (jnp.sum(oh * jnp.broadcast_to(blk_start, (t, e)),
                            axis=1, keepdims=True) * _TB + rank)
    jb = lax.broadcasted_iota(jnp.int32, (_SP, e), 0) - 1
    ge = (jb >= jnp.broadcast_to(blk_start, (_SP, e))).astype(jnp.int32)
    be = jnp.clip(jnp.sum(ge, axis=1, keepdims=True) - 1, 0, e - 1)
    row = lax.broadcasted_iota(jnp.int32, (_SP, 1), 0)
    sp_ref[...] = jnp.where(row == 0,
                            jnp.broadcast_to(total, (_SP, 1)), be)


def _moe_body(sp_ref, xp_ref, wg_ref, wu_ref, wd_ref, out_ref):
    @pl.when(pl.program_id(0) < sp_ref[0, 0])
    def _():
        xb = xp_ref[...]
        g = lax.dot_general(xb, wg_ref[0], (((1,), (1,)), ((), ())),
                            preferred_element_type=jnp.float32)
        u = lax.dot_general(xb, wu_ref[0], (((1,), (1,)), ((), ())),
                            preferred_element_type=jnp.float32)
        hmid = jnp.square(jnp.maximum(g, 0.0)) * u
        out_ref[...] = lax.dot_general(hmid, wd_ref[0],
                                       (((1,), (1,)), ((), ())),
                                       preferred_element_type=jnp.float32)


def kernel(x, router_w, Wg, Wu, Wd):
    b, s, hdim = x.shape
    e, d, _ = Wg.shape
    t = b * s
    xf = x.reshape(t, hdim)

    nb = t // _TB + e
    p = nb * _TB
    dst, sp = pl.pallas_call(
        _router_book_body,
        out_shape=(jax.ShapeDtypeStruct((t, 1), jnp.int32),
                   jax.ShapeDtypeStruct((_SP, 1), jnp.int32)),
    )(xf, router_w)
    bpw = t // _NW
    dst2 = dst.reshape(_NW, bpw)

    mesh = plsc.VectorSubcoreMesh(core_axis_name="c", subcore_axis_name="s")

    @functools.partial(
        pl.kernel, mesh=mesh,
        out_type=jax.ShapeDtypeStruct((p, hdim), jnp.float32),
        scratch_types=[
            pltpu.VMEM((bpw,), jnp.int32),
            pltpu.VMEM((bpw, hdim), jnp.float32),
            pltpu.SemaphoreType.DMA,
        ],
    )
    def _dispatch(xf_hbm, dst_hbm, xp_hbm, idx_v, rows_v, sem):
        wid = lax.axis_index("s") * _NC + lax.axis_index("c")
        pltpu.sync_copy(dst_hbm.at[wid], idx_v)
        pltpu.sync_copy(xf_hbm.at[pl.ds(wid * bpw, bpw)], rows_v)
        pltpu.async_copy(rows_v, xp_hbm.at[idx_v], sem).wait()

    xp = _dispatch(xf, dst2)

    grid_spec = pltpu.PrefetchScalarGridSpec(
        num_scalar_prefetch=1,
        grid=(nb,),
        in_specs=[
            pl.BlockSpec((_TB, hdim),
                         lambda i, sp: (jnp.minimum(i, sp[0, 0] - 1), 0)),
            pl.BlockSpec((1, d, hdim), lambda i, sp: (sp[i + 1, 0], 0, 0)),
            pl.BlockSpec((1, d, hdim), lambda i, sp: (sp[i + 1, 0], 0, 0)),
            pl.BlockSpec((1, hdim, d), lambda i, sp: (sp[i + 1, 0], 0, 0)),
        ],
        out_specs=pl.BlockSpec((_TB, hdim),
                               lambda i, sp: (jnp.minimum(i, sp[0, 0] - 1), 0)),
    )
    out_p = pl.pallas_call(
        _moe_body,
        grid_spec=grid_spec,
        out_shape=jax.ShapeDtypeStruct((p, hdim), jnp.float32),
    )(sp, xp, Wg, Wu, Wd)

    @functools.partial(
        pl.kernel, mesh=mesh,
        out_type=jax.ShapeDtypeStruct((t, hdim), jnp.float32),
        scratch_types=[
            pltpu.VMEM((bpw,), jnp.int32),
            pltpu.VMEM((bpw, hdim), jnp.float32),
            pltpu.SemaphoreType.DMA,
        ],
    )
    def _combine(outp_hbm, dst_hbm, out_hbm, idx_v, rows_v, sem):
        wid = lax.axis_index("s") * _NC + lax.axis_index("c")
        pltpu.sync_copy(dst_hbm.at[wid], idx_v)
        pltpu.async_copy(outp_hbm.at[idx_v], rows_v, sem).wait()
        pltpu.sync_copy(rows_v, out_hbm.at[pl.ds(wid * bpw, bpw)])

    out = _combine(out_p, dst2)
    return out.reshape(b, s, hdim)
